# baseline (speedup 1.0000x reference)
import functools

import jax
import jax.numpy as jnp
from jax import lax
from jax.experimental import pallas as pl
from jax.experimental.pallas import tpu as pltpu
from jax.experimental.pallas import tpu_sc as plsc

EPS = 1e-9
LANES = 16
CHUNK = 32
NBUF = 4
RGROUP = 4


def _make_sc_kernel(N, D, n_workers):
    b_per_w = N // n_workers
    n_chunks = b_per_w // CHUNK
    n_vec = D // LANES
    n_groups = CHUNK // RGROUP
    mesh = plsc.VectorSubcoreMesh(core_axis_name="c", subcore_axis_name="s")

    @functools.partial(
        pl.kernel,
        mesh=mesh,
        out_type=jax.ShapeDtypeStruct((N, D), jnp.float32),
        scratch_types=(
            [pltpu.VMEM((b_per_w,), jnp.int32)]
            + [pltpu.VMEM((CHUNK, D), jnp.float32) for _ in range(NBUF)]
            + [pltpu.VMEM((D,), jnp.float32)] * 2
            + [pltpu.SemaphoreType.DMA] * (2 * NBUF)
        ),
    )
    def emb_ln(w_hbm, idx_hbm, g_hbm, b_hbm, out_hbm, *scratch):
        idx_v = scratch[0]
        bufs = scratch[1:1 + NBUF]
        g_v, b_v = scratch[1 + NBUF:3 + NBUF]
        gsems = scratch[3 + NBUF:3 + 2 * NBUF]
        ssems = scratch[3 + 2 * NBUF:3 + 3 * NBUF]

        num_c = lax.axis_size("c")
        wid = lax.axis_index("s") * num_c + lax.axis_index("c")
        base = wid * b_per_w

        if True:
            return
        pltpu.sync_copy(idx_hbm.at[pl.ds(base, b_per_w)], idx_v)

        def gather_copy(c, b):
            return pltpu.make_async_copy(
                w_hbm.at[idx_v.at[pl.ds(c * CHUNK, CHUNK)]], bufs[b],
                gsems[b])

        def scatter_copy(c, b):
            return pltpu.make_async_copy(
                bufs[b], out_hbm.at[pl.ds(base + c * CHUNK, CHUNK)],
                ssems[b])

        lane_iota = lax.iota(jnp.int32, LANES)
        gdn = lax.GatherDimensionNumbers(
            offset_dims=(), collapsed_slice_dims=(0,), start_index_map=(0,))

        def lane_total(x):
            for k in (8, 4, 2, 1):
                perm = lane_iota ^ k
                x = x + lax.gather(
                    x, perm[:, None], gdn, slice_sizes=(1,),
                    mode=lax.GatherScatterMode.PROMISE_IN_BOUNDS)
            return x

        def compute_fast(buf):
            @plsc.parallel_loop(0, n_groups)
            def fgroup_body(gi):
                r0 = gi * RGROUP
                ss = [jnp.zeros((LANES,), jnp.float32)
                      for _ in range(RGROUP)]
                sqs = [jnp.zeros((LANES,), jnp.float32)
                       for _ in range(RGROUP)]
                for j in range(n_vec):
                    for r in range(RGROUP):
                        x = buf[r0 + r, pl.ds(j * LANES, LANES)]
                        ss[r] = ss[r] + x
                        sqs[r] = sqs[r] + x * x
                mrs = []
                ys = []
                for r in range(RGROUP):
                    mean = lane_total(ss[r]) * (1.0 / D)
                    var = lane_total(sqs[r]) * (1.0 / D) - mean * mean
                    vv = var + EPS
                    bits = lax.bitcast_convert_type(vv, jnp.int32)
                    bits = jnp.int32(0x5F3759DF) - (bits >> 1)
                    y = lax.bitcast_convert_type(bits, jnp.float32)
                    half = vv * 0.5
                    y = y * (1.5 - half * y * y)
                    y = y * (1.5 - half * y * y)
                    mrs.append(mean * y)
                    ys.append(y)
                for j in range(n_vec):
                    sl = pl.ds(j * LANES, LANES)
                    for r in range(RGROUP):
                        x = buf[r0 + r, sl]
                        buf[r0 + r, sl] = x * ys[r] - mrs[r]

        def compute_gen(buf):
            def grow_body(r, _):
                def acc(j, carry):
                    s, sq = carry
                    x = buf[r, pl.ds(j * LANES, LANES)]
                    return s + x, sq + x * x

                zeros = jnp.zeros((LANES,), jnp.float32)
                s, sq = lax.fori_loop(0, n_vec, acc, (zeros, zeros))
                mean = lane_total(s) * (1.0 / D)
                var = lane_total(sq) * (1.0 / D) - mean * mean
                vv = var + EPS
                bits = lax.bitcast_convert_type(vv, jnp.int32)
                bits = jnp.int32(0x5F3759DF) - (bits >> 1)
                y = lax.bitcast_convert_type(bits, jnp.float32)
                half = vv * 0.5
                y = y * (1.5 - half * y * y)
                y = y * (1.5 - half * y * y)
                mr = mean * y

                def norm(j, _):
                    sl = pl.ds(j * LANES, LANES)
                    x = buf[r, sl]
                    buf[r, sl] = (x * y - mr) * g_v[sl] + b_v[sl]
                    return 0

                lax.fori_loop(0, n_vec, norm, 0)
                return 0

            lax.fori_loop(0, CHUNK, grow_body, 0)

        plain_acc = jnp.ones((LANES,), jnp.bool_)
        for j in range(n_vec):
            sl = pl.ds(j * LANES, LANES)
            plain_acc = (plain_acc & (g_v[sl] == 1.0)
                         & (b_v[sl] == 0.0))
        plain_f = jnp.where(plain_acc, 1.0, 0.0).astype(jnp.float32)
        is_plain = lane_total(plain_f)[0] == jnp.float32(LANES)

        LOOKAHEAD = 2

        for c in range(min(LOOKAHEAD, n_chunks)):
            gather_copy(c, c).start()

        pltpu.sync_copy(g_hbm, g_v)
        pltpu.sync_copy(b_hbm, b_v)

        def run_pipeline(compute_fn):
            def round_body(rd, _):
                for b in range(NBUF):
                    c = rd * NBUF + b
                    nb = (b + LOOKAHEAD) % NBUF

                    @pl.when((c >= LOOKAHEAD) & (c + LOOKAHEAD < n_chunks))
                    def _():
                        scatter_copy(c - LOOKAHEAD, nb).wait()

                    @pl.when(c + LOOKAHEAD < n_chunks)
                    def _():
                        gather_copy(c + LOOKAHEAD, nb).start()

                    gather_copy(c, b).wait()
                    compute_fn(bufs[b])
                    scatter_copy(c, b).start()
                return 0

            lax.fori_loop(0, n_chunks // NBUF, round_body, 0)

            for b in range(min(NBUF, n_chunks)):
                c_last = n_chunks - NBUF + b
                scatter_copy(c_last, c_last % NBUF).wait()

        @pl.when(is_plain)
        def _():
            run_pipeline(compute_fast)

        @pl.when(jnp.logical_not(is_plain))
        def _():
            run_pipeline(compute_gen)

    return emb_ln


def kernel(input_ids, weight, ln_gamma, ln_beta):
    bt, seq = input_ids.shape
    vocab, d = weight.shape
    n = bt * seq
    info = plsc.get_sparse_core_info()
    n_workers = info.num_cores * info.num_subcores
    idx = input_ids.reshape(n).astype(jnp.int32)
    emb_ln = _make_sc_kernel(n, d, n_workers)
    out = emb_ln(weight, idx, ln_gamma, ln_beta)
    return out.reshape(bt, seq, d)


# speedup vs baseline: 1.0089x; 1.0089x over previous
import functools

import jax
import jax.numpy as jnp
from jax import lax
from jax.experimental import pallas as pl
from jax.experimental.pallas import tpu as pltpu
from jax.experimental.pallas import tpu_sc as plsc

EPS = 1e-9
LANES = 16
CHUNK = 32
NBUF = 4
RGROUP = 4


def _make_sc_kernel(N, D, n_workers):
    b_per_w = N // n_workers
    n_chunks = b_per_w // CHUNK
    n_vec = D // LANES
    n_groups = CHUNK // RGROUP
    mesh = plsc.VectorSubcoreMesh(core_axis_name="c", subcore_axis_name="s")

    @functools.partial(
        pl.kernel,
        mesh=mesh,
        out_type=jax.ShapeDtypeStruct((N, D), jnp.float32),
        scratch_types=(
            [pltpu.VMEM((b_per_w,), jnp.int32)]
            + [pltpu.VMEM((CHUNK, D), jnp.float32) for _ in range(NBUF)]
            + [pltpu.VMEM((D,), jnp.float32)] * 2
            + [pltpu.SemaphoreType.DMA] * (2 * NBUF)
        ),
    )
    def emb_ln(w_hbm, idx_hbm, g_hbm, b_hbm, out_hbm, *scratch):
        idx_v = scratch[0]
        bufs = scratch[1:1 + NBUF]
        g_v, b_v = scratch[1 + NBUF:3 + NBUF]
        gsems = scratch[3 + NBUF:3 + 2 * NBUF]
        ssems = scratch[3 + 2 * NBUF:3 + 3 * NBUF]

        num_c = lax.axis_size("c")
        wid = lax.axis_index("s") * num_c + lax.axis_index("c")
        base = wid * b_per_w

        pltpu.sync_copy(idx_hbm.at[pl.ds(base, b_per_w)], idx_v)

        def gather_copy(c, b):
            return pltpu.make_async_copy(
                w_hbm.at[idx_v.at[pl.ds(c * CHUNK, CHUNK)]], bufs[b],
                gsems[b])

        def scatter_copy(c, b):
            return pltpu.make_async_copy(
                bufs[b], out_hbm.at[pl.ds(base + c * CHUNK, CHUNK)],
                ssems[b])

        lane_iota = lax.iota(jnp.int32, LANES)
        gdn = lax.GatherDimensionNumbers(
            offset_dims=(), collapsed_slice_dims=(0,), start_index_map=(0,))

        def lane_total(x):
            for k in (8, 4, 2, 1):
                perm = lane_iota ^ k
                x = x + lax.gather(
                    x, perm[:, None], gdn, slice_sizes=(1,),
                    mode=lax.GatherScatterMode.PROMISE_IN_BOUNDS)
            return x

        def compute_fast(buf):
            @plsc.parallel_loop(0, n_groups)
            def fgroup_body(gi):
                r0 = gi * RGROUP
                ss = [jnp.zeros((LANES,), jnp.float32)
                      for _ in range(RGROUP)]
                sqs = [jnp.zeros((LANES,), jnp.float32)
                       for _ in range(RGROUP)]
                for j in range(n_vec):
                    for r in range(RGROUP):
                        x = buf[r0 + r, pl.ds(j * LANES, LANES)]
                        ss[r] = ss[r] + x
                        sqs[r] = sqs[r] + x * x
                mrs = []
                ys = []
                for r in range(RGROUP):
                    mean = lane_total(ss[r]) * (1.0 / D)
                    var = lane_total(sqs[r]) * (1.0 / D) - mean * mean
                    vv = var + EPS
                    bits = lax.bitcast_convert_type(vv, jnp.int32)
                    bits = jnp.int32(0x5F3759DF) - (bits >> 1)
                    y = lax.bitcast_convert_type(bits, jnp.float32)
                    half = vv * 0.5
                    y = y * (1.5 - half * y * y)
                    y = y * (1.5 - half * y * y)
                    mrs.append(mean * y)
                    ys.append(y)
                for j in range(n_vec):
                    sl = pl.ds(j * LANES, LANES)
                    for r in range(RGROUP):
                        x = buf[r0 + r, sl]
                        buf[r0 + r, sl] = x * ys[r] - mrs[r]

        def compute_gen(buf):
            def grow_body(r, _):
                def acc(j, carry):
                    s, sq = carry
                    x = buf[r, pl.ds(j * LANES, LANES)]
                    return s + x, sq + x * x

                zeros = jnp.zeros((LANES,), jnp.float32)
                s, sq = lax.fori_loop(0, n_vec, acc, (zeros, zeros))
                mean = lane_total(s) * (1.0 / D)
                var = lane_total(sq) * (1.0 / D) - mean * mean
                vv = var + EPS
                bits = lax.bitcast_convert_type(vv, jnp.int32)
                bits = jnp.int32(0x5F3759DF) - (bits >> 1)
                y = lax.bitcast_convert_type(bits, jnp.float32)
                half = vv * 0.5
                y = y * (1.5 - half * y * y)
                y = y * (1.5 - half * y * y)
                mr = mean * y

                def norm(j, _):
                    sl = pl.ds(j * LANES, LANES)
                    x = buf[r, sl]
                    buf[r, sl] = (x * y - mr) * g_v[sl] + b_v[sl]
                    return 0

                lax.fori_loop(0, n_vec, norm, 0)
                return 0

            lax.fori_loop(0, CHUNK, grow_body, 0)

        LOOKAHEAD = 2

        for c in range(min(LOOKAHEAD, n_chunks)):
            gather_copy(c, c).start()

        pltpu.sync_copy(g_hbm, g_v)
        pltpu.sync_copy(b_hbm, b_v)

        plain_acc = jnp.ones((LANES,), jnp.bool_)
        for j in range(n_vec):
            sl = pl.ds(j * LANES, LANES)
            plain_acc = (plain_acc & (g_v[sl] == 1.0)
                         & (b_v[sl] == 0.0))
        plain_f = jnp.where(plain_acc, 1.0, 0.0).astype(jnp.float32)
        is_plain = lane_total(plain_f)[0] == jnp.float32(LANES)

        def run_pipeline(compute_fn):
            def round_body(rd, _):
                for b in range(NBUF):
                    c = rd * NBUF + b
                    nb = (b + LOOKAHEAD) % NBUF

                    @pl.when((c >= LOOKAHEAD) & (c + LOOKAHEAD < n_chunks))
                    def _():
                        scatter_copy(c - LOOKAHEAD, nb).wait()

                    @pl.when(c + LOOKAHEAD < n_chunks)
                    def _():
                        gather_copy(c + LOOKAHEAD, nb).start()

                    gather_copy(c, b).wait()
                    compute_fn(bufs[b])
                    scatter_copy(c, b).start()
                return 0

            lax.fori_loop(0, n_chunks // NBUF, round_body, 0)

            for b in range(min(NBUF, n_chunks)):
                c_last = n_chunks - NBUF + b
                scatter_copy(c_last, c_last % NBUF).wait()

        @pl.when(is_plain)
        def _():
            run_pipeline(compute_fast)

        @pl.when(jnp.logical_not(is_plain))
        def _():
            run_pipeline(compute_gen)

    return emb_ln


def kernel(input_ids, weight, ln_gamma, ln_beta):
    bt, seq = input_ids.shape
    vocab, d = weight.shape
    n = bt * seq
    info = plsc.get_sparse_core_info()
    n_workers = info.num_cores * info.num_subcores
    idx = input_ids.reshape(n).astype(jnp.int32)
    emb_ln = _make_sc_kernel(n, d, n_workers)
    out = emb_ln(weight, idx, ln_gamma, ln_beta)
    return out.reshape(bt, seq, d)
